# baseline (speedup 1.0000x reference)
import functools

import jax
import jax.numpy as jnp
from jax import lax
from jax.experimental import pallas as pl
from jax.experimental.pallas import tpu as pltpu
from jax.experimental.pallas import tpu_sc as plsc

N = 10000
E = 320000
D = 128
EPS = 1e-3
RS = (1.0 + EPS) ** -0.5

NC = 2
NS = 16
NW = NC * NS
CH = 80
NB = 4
NI = 8
NCH = 256
GCH = NS * NCH
EPAD = GCH * CH
NP = 10112
ZR = NP // NS


def _gelu(x):
    return 0.5 * x * (1.0 + lax.erf(x * (2.0 ** -0.5)))


def _prep_body(x_ref, g_ref, bt_ref, w_ref, b_ref, o_ref):
    h = x_ref[...] * (g_ref[...] * RS) + bt_ref[...]
    o_ref[...] = _gelu(
        jnp.dot(h, w_ref[...], preferred_element_type=jnp.float32) + b_ref[...])


def _prep_msgs(x, gamma1, beta1, W_prep, b_prep):
    BR = 512
    grid = (pl.cdiv(N, BR),)
    return pl.pallas_call(
        _prep_body,
        grid=grid,
        in_specs=[
            pl.BlockSpec((BR, D), lambda i: (i, 0)),
            pl.BlockSpec((D,), lambda i: (0,)),
            pl.BlockSpec((D,), lambda i: (0,)),
            pl.BlockSpec((D, D), lambda i: (0, 0)),
            pl.BlockSpec((D,), lambda i: (0,)),
        ],
        out_specs=pl.BlockSpec((BR, D), lambda i: (i, 0)),
        out_shape=jax.ShapeDtypeStruct((N, D), jnp.float32),
    )(x, gamma1, beta1, W_prep, b_prep)


def _sc_body(m_hbm, src_hbm, dst_hbm, w_hbm,
             sums_out, cnts_out,
             srcb, dstb, wb, rows_v, ones_v, zcnt, sums_sh, cnts_sh,
             gsems, ssems, csems, isems):
    c_ax = lax.axis_index("c")
    s = lax.axis_index("s")
    is0 = c_ax == 0
    is1 = c_ax == 1
    base = s * NCH

    for i in range(CH // 16):
        ones_v[pl.ds(i * 16, 16)] = jnp.ones((16,), jnp.float32)

    scope_init = jax.named_scope("sc_zinit")
    scope_init.__enter__()
    zv = jnp.zeros((16,), jnp.float32)

    def zrow(b, _):
        for k in range(D // 16):
            rows_v[0, b, pl.ds(k * 16, 16)] = zv
        return 0

    lax.fori_loop(0, CH, zrow, 0, unroll=2)

    def zc(i, _):
        zcnt[pl.ds(i * 16, 16)] = zv
        return 0

    lax.fori_loop(0, 640 // 16, zc, 0, unroll=2)

    for t in range(8):
        pltpu.sync_copy(rows_v.at[0, pl.ds(0, 72)],
                        sums_sh.at[pl.ds(s * ZR + t * 72, 72)])
    pltpu.sync_copy(rows_v.at[0, pl.ds(0, 56)],
                    sums_sh.at[pl.ds(s * ZR + 576, 56)])
    pltpu.sync_copy(zcnt.at[pl.ds(0, ZR)], cnts_sh.at[pl.ds(s * ZR, ZR)])

    plsc.subcore_barrier()
    scope_init.__exit__(None, None, None)
    scope_main = jax.named_scope("sc_mainloop")
    scope_main.__enter__()

    def idx_fetch_start(c, sem):
        r = c % NI
        off = (base + c) * CH
        pltpu.async_copy(dst_hbm.at[pl.ds(off, CH)], dstb.at[r], sem)
        pltpu.async_copy(src_hbm.at[pl.ds(off, CH)], srcb.at[r], sem)
        pltpu.async_copy(w_hbm.at[pl.ds(off, CH)], wb.at[r], sem)

    def idx_fetch_wait(c, sem):
        r = c % NI
        off = (base + c) * CH
        pltpu.make_async_copy(dst_hbm.at[pl.ds(off, CH)], dstb.at[r], sem).wait()
        pltpu.make_async_copy(src_hbm.at[pl.ds(off, CH)], srcb.at[r],
                              sem).wait()
        pltpu.make_async_copy(w_hbm.at[pl.ds(off, CH)], wb.at[r], sem).wait()

    def gather_start(c, b, sem):
        pltpu.async_copy(m_hbm.at[srcb.at[c % NI]], rows_v.at[b], sem)

    def row_scatter_wait(c, j):
        pltpu.make_async_copy(rows_v.at[j], sums_sh.at[dstb.at[c % NI]],
                              ssems[j]).wait()

    def cnt_scatter_wait(c, j):
        pltpu.make_async_copy(ones_v, cnts_sh.at[dstb.at[c % NI]],
                              csems[j]).wait()

    def mul_chunk(c, j):
        r = c % NI

        def grp(gi, _):
            wvec = wb[r, pl.ds(gi * 16, 16)]
            for t in range(16):
                wv = wvec[t]
                be = gi * 16 + t
                for k in range(D // 16):
                    rows_v[j, be, pl.ds(k * 16, 16)] = (
                        rows_v[j, be, pl.ds(k * 16, 16)] * wv)
            return 0
        lax.fori_loop(0, CH // 16, grp, 0, unroll=2)

    @pl.when(is0)
    def _():
        idx_fetch_start(0, isems[0])
        idx_fetch_start(1, isems[1])
        idx_fetch_start(2, isems[2])
        idx_fetch_wait(0, isems[0])
        gather_start(0, 0, gsems[0])
        idx_fetch_wait(1, isems[1])
        gather_start(1, 1, gsems[1])

        def outer(g, _):
            for j in range(NB):
                c = g * NB + j

                @pl.when(c + 3 < NCH)
                def _():
                    idx_fetch_start(c + 3, isems[(j + 3) % NB])

                @pl.when(c + 2 < NCH)
                def _():
                    b2 = (j + 2) % NB

                    @pl.when(c >= 2)
                    def _():
                        row_scatter_wait(c - 2, b2)
                    idx_fetch_wait(c + 2, isems[(j + 2) % NB])
                    gather_start(c + 2, b2, gsems[b2])

                pltpu.make_async_copy(m_hbm.at[srcb.at[c % NI]],
                                      rows_v.at[j], gsems[j]).wait()
                mul_chunk(c, j)
                pltpu.async_copy(rows_v.at[j], sums_sh.at[dstb.at[c % NI]],
                                 ssems[j], add=True)
            return 0

        lax.fori_loop(0, NCH // NB, outer, 0)
        for j in range(NB):
            row_scatter_wait(NCH - NB + j, j)

    @pl.when(is1)
    def _():
        def cfetch(c, sem):
            r = c % NI
            pltpu.async_copy(dst_hbm.at[pl.ds((base + c) * CH, CH)],
                             dstb.at[r], sem)

        def cfetch_wait(c, sem):
            r = c % NI
            pltpu.make_async_copy(dst_hbm.at[pl.ds((base + c) * CH, CH)],
                                  dstb.at[r], sem).wait()

        cfetch(0, isems[0])
        cfetch(1, isems[1])
        cfetch(2, isems[2])

        def outer1(g, _):
            for j in range(NB):
                c = g * NB + j

                @pl.when(c + 3 < NCH)
                def _():
                    cfetch(c + 3, isems[(j + 3) % NB])

                @pl.when(c >= 2)
                def _():
                    cnt_scatter_wait(c - 2, (j + 2) % NB)
                cfetch_wait(c, isems[j])
                pltpu.async_copy(ones_v, cnts_sh.at[dstb.at[c % NI]],
                                 csems[j], add=True)
            return 0

        lax.fori_loop(0, NCH // NB, outer1, 0)
        cnt_scatter_wait(NCH - 2, (NCH - 2) % NB)
        cnt_scatter_wait(NCH - 1, (NCH - 1) % NB)

    plsc.subcore_barrier()
    scope_main.__exit__(None, None, None)
    scope_wb = jax.named_scope("sc_wback")
    scope_wb.__enter__()

    @pl.when(is0)
    def _():
        pltpu.sync_copy(sums_sh.at[pl.ds(s * ZR, ZR)],
                        sums_out.at[pl.ds(s * ZR, ZR)])

    @pl.when(is1 & (s == 0))
    def _():
        pltpu.sync_copy(cnts_sh, cnts_out)
    scope_wb.__exit__(None, None, None)


def _sc_aggregate(msgs, src, dst, w):
    mesh = plsc.VectorSubcoreMesh(core_axis_name="c", subcore_axis_name="s")
    kern = pl.kernel(
        _sc_body,
        out_type=[
            jax.ShapeDtypeStruct((NP, D), jnp.float32),
            jax.ShapeDtypeStruct((NP,), jnp.float32),
        ],
        mesh=mesh,
        scratch_types=[
            pltpu.VMEM((NI, CH), jnp.int32),
            pltpu.VMEM((NI, CH), jnp.int32),
            pltpu.VMEM((NI, CH), jnp.float32),
            pltpu.VMEM((NB, CH, D), jnp.float32),
            pltpu.VMEM((CH,), jnp.float32),
            pltpu.VMEM((640,), jnp.float32),
            pltpu.VMEM_SHARED((NP, D), jnp.float32),
            pltpu.VMEM_SHARED((NP,), jnp.float32),
            [pltpu.SemaphoreType.DMA] * NB,
            [pltpu.SemaphoreType.DMA] * NB,
            [pltpu.SemaphoreType.DMA] * NB,
            [pltpu.SemaphoreType.DMA] * NB,
        ],
    )
    return kern(msgs, src, dst, w)


def _upd_body(x_ref, s_ref, c_ref, g_ref, bt_ref, w_ref, b_ref, o_ref):
    agg = s_ref[...] / jnp.maximum(c_ref[...], 1.0)[:, None]
    hx = x_ref[...] * (g_ref[pl.ds(0, D)] * RS) + bt_ref[pl.ds(0, D)]
    ha = agg * (g_ref[pl.ds(D, D)] * RS) + bt_ref[pl.ds(D, D)]
    acc = jnp.dot(hx, w_ref[pl.ds(0, D), :], preferred_element_type=jnp.float32)
    acc += jnp.dot(ha, w_ref[pl.ds(D, D), :], preferred_element_type=jnp.float32)
    o_ref[...] = _gelu(acc + b_ref[...])


def _update(x, sums, cnts, gamma2, beta2, W_upd, b_upd):
    BR = 512
    grid = (pl.cdiv(N, BR),)
    return pl.pallas_call(
        _upd_body,
        grid=grid,
        in_specs=[
            pl.BlockSpec((BR, D), lambda i: (i, 0)),
            pl.BlockSpec((BR, D), lambda i: (i, 0)),
            pl.BlockSpec((BR,), lambda i: (i,)),
            pl.BlockSpec((2 * D,), lambda i: (0,)),
            pl.BlockSpec((2 * D,), lambda i: (0,)),
            pl.BlockSpec((2 * D, D), lambda i: (0, 0)),
            pl.BlockSpec((D,), lambda i: (0,)),
        ],
        out_specs=pl.BlockSpec((BR, D), lambda i: (i, 0)),
        out_shape=jax.ShapeDtypeStruct((N, D), jnp.float32),
    )(x, sums, cnts, gamma2, beta2, W_upd, b_upd)


@jax.jit
def kernel(node_representation, edges, edge_weights, gamma1, beta1, W_prep,
           b_prep, gamma2, beta2, W_upd, b_upd):
    x = node_representation
    dst = edges[0].astype(jnp.int32)
    src = edges[1].astype(jnp.int32)
    w = edge_weights

    npad = EPAD - E
    src = jnp.concatenate([src, jnp.zeros((npad,), jnp.int32)])
    dst = jnp.concatenate([dst, jnp.full((npad,), N, jnp.int32)])
    w = jnp.concatenate([w, jnp.zeros((npad,), jnp.float32)])

    msgs = _prep_msgs(x, gamma1, beta1, W_prep, b_prep)
    sums, cnts = _sc_aggregate(msgs, src, dst, w)
    return _update(x, sums, cnts, gamma2, beta2, W_upd, b_upd)


# speedup vs baseline: 1.3772x; 1.3772x over previous
import functools

import jax
import jax.numpy as jnp
from jax import lax
from jax.experimental import pallas as pl
from jax.experimental.pallas import tpu as pltpu
from jax.experimental.pallas import tpu_sc as plsc

N = 10000
E = 320000
D = 128
EPS = 1e-3
RS = (1.0 + EPS) ** -0.5

NC = 2
NS = 16
NW = NC * NS
CH = 80
NB = 4
NI = 8
NCH0 = 228
NCH1 = 28
GCH = NS * (NCH0 + NCH1)
EPAD = GCH * CH
NP = 10112
ZR = NP // NS


def _gelu(x):
    return 0.5 * x * (1.0 + lax.erf(x * (2.0 ** -0.5)))


def _prep_body(x_ref, g_ref, bt_ref, w_ref, b_ref, o_ref):
    h = x_ref[...] * (g_ref[...] * RS) + bt_ref[...]
    o_ref[...] = _gelu(
        jnp.dot(h, w_ref[...], preferred_element_type=jnp.float32) + b_ref[...])


def _prep_msgs(x, gamma1, beta1, W_prep, b_prep):
    BR = 512
    grid = (pl.cdiv(N, BR),)
    return pl.pallas_call(
        _prep_body,
        grid=grid,
        in_specs=[
            pl.BlockSpec((BR, D), lambda i: (i, 0)),
            pl.BlockSpec((D,), lambda i: (0,)),
            pl.BlockSpec((D,), lambda i: (0,)),
            pl.BlockSpec((D, D), lambda i: (0, 0)),
            pl.BlockSpec((D,), lambda i: (0,)),
        ],
        out_specs=pl.BlockSpec((BR, D), lambda i: (i, 0)),
        out_shape=jax.ShapeDtypeStruct((N, D), jnp.float32),
    )(x, gamma1, beta1, W_prep, b_prep)


def _sc_body(m_hbm, src_hbm, dst_hbm, w_hbm,
             sums_out, cnts_out,
             srcb, dstb, wb, rows_v, ones_v, zcnt, sums_sh, cnts_sh,
             gsems, ssems, csems, isems):
    c_ax = lax.axis_index("c")
    s = lax.axis_index("s")
    nch = jnp.where(c_ax == 0, NCH0, NCH1)
    base = jnp.where(c_ax == 0, s * NCH0, NS * NCH0 + s * NCH1)

    for i in range(CH // 16):
        ones_v[pl.ds(i * 16, 16)] = jnp.ones((16,), jnp.float32)

    scope_init = jax.named_scope("sc_zinit")
    scope_init.__enter__()
    zv = jnp.zeros((16,), jnp.float32)

    def zrow(b, _):
        for k in range(D // 16):
            rows_v[0, b, pl.ds(k * 16, 16)] = zv
        return 0

    lax.fori_loop(0, CH, zrow, 0, unroll=2)

    def zc(i, _):
        zcnt[pl.ds(i * 16, 16)] = zv
        return 0

    lax.fori_loop(0, 640 // 16, zc, 0, unroll=2)

    for t in range(8):
        pltpu.sync_copy(rows_v.at[0, pl.ds(0, 72)],
                        sums_sh.at[pl.ds(s * ZR + t * 72, 72)])
    pltpu.sync_copy(rows_v.at[0, pl.ds(0, 56)],
                    sums_sh.at[pl.ds(s * ZR + 576, 56)])
    pltpu.sync_copy(zcnt.at[pl.ds(0, ZR)], cnts_sh.at[pl.ds(s * ZR, ZR)])

    plsc.subcore_barrier()
    scope_init.__exit__(None, None, None)
    scope_main = jax.named_scope("sc_mainloop")
    scope_main.__enter__()

    def idx_fetch_start(c, sem):
        r = c % NI
        off = (base + c) * CH
        pltpu.async_copy(dst_hbm.at[pl.ds(off, CH)], dstb.at[r], sem)
        pltpu.async_copy(src_hbm.at[pl.ds(off, CH)], srcb.at[r], sem)
        pltpu.async_copy(w_hbm.at[pl.ds(off, CH)], wb.at[r], sem)

    def idx_fetch_wait(c, sem):
        r = c % NI
        off = (base + c) * CH
        pltpu.make_async_copy(dst_hbm.at[pl.ds(off, CH)], dstb.at[r], sem).wait()
        pltpu.make_async_copy(src_hbm.at[pl.ds(off, CH)], srcb.at[r],
                              sem).wait()
        pltpu.make_async_copy(w_hbm.at[pl.ds(off, CH)], wb.at[r], sem).wait()

    def gather_start(c, b, sem):
        pltpu.async_copy(m_hbm.at[srcb.at[c % NI]], rows_v.at[b], sem)

    def row_scatter_wait(c, j):
        pltpu.make_async_copy(rows_v.at[j], sums_sh.at[dstb.at[c % NI]],
                              ssems[j]).wait()

    def cnt_scatter_wait(c, j):
        pltpu.make_async_copy(ones_v, cnts_sh.at[dstb.at[c % NI]],
                              csems[j]).wait()

    def mul_chunk(c, j):
        r = c % NI

        def grp(gi, _):
            wvec = wb[r, pl.ds(gi * 16, 16)]
            for t in range(16):
                wv = wvec[t]
                be = gi * 16 + t
                for k in range(D // 16):
                    rows_v[j, be, pl.ds(k * 16, 16)] = (
                        rows_v[j, be, pl.ds(k * 16, 16)] * wv)
            return 0
        lax.fori_loop(0, CH // 16, grp, 0, unroll=2)

    idx_fetch_start(0, isems[0])
    idx_fetch_start(1, isems[1])
    idx_fetch_start(2, isems[2])
    idx_fetch_wait(0, isems[0])
    gather_start(0, 0, gsems[0])
    idx_fetch_wait(1, isems[1])
    gather_start(1, 1, gsems[1])

    def outer(g, _):
        for j in range(NB):
            c = g * NB + j

            @pl.when(c + 3 < nch)
            def _():
                idx_fetch_start(c + 3, isems[(j + 3) % NB])

            @pl.when(c + 2 < nch)
            def _():
                b2 = (j + 2) % NB

                @pl.when(c >= 2)
                def _():
                    row_scatter_wait(c - 2, b2)
                    cnt_scatter_wait(c - 2, b2)
                idx_fetch_wait(c + 2, isems[(j + 2) % NB])
                gather_start(c + 2, b2, gsems[b2])

            pltpu.make_async_copy(m_hbm.at[srcb.at[c % NI]],
                                  rows_v.at[j], gsems[j]).wait()
            mul_chunk(c, j)
            pltpu.async_copy(rows_v.at[j], sums_sh.at[dstb.at[c % NI]],
                             ssems[j], add=True)
            pltpu.async_copy(ones_v, cnts_sh.at[dstb.at[c % NI]],
                             csems[j], add=True)
        return 0

    lax.fori_loop(0, nch // NB, outer, 0)
    for j in range(NB):
        row_scatter_wait(nch - NB + j, j)
        cnt_scatter_wait(nch - NB + j, j)

    plsc.subcore_barrier()
    scope_main.__exit__(None, None, None)
    scope_wb = jax.named_scope("sc_wback")
    scope_wb.__enter__()

    pltpu.sync_copy(sums_sh.at[pl.ds(s * ZR, ZR)],
                    sums_out.at[c_ax, pl.ds(s * ZR, ZR)])

    @pl.when(s == 0)
    def _():
        pltpu.sync_copy(cnts_sh, cnts_out.at[c_ax])
    scope_wb.__exit__(None, None, None)


def _sc_aggregate(msgs, src, dst, w):
    mesh = plsc.VectorSubcoreMesh(core_axis_name="c", subcore_axis_name="s")
    kern = pl.kernel(
        _sc_body,
        out_type=[
            jax.ShapeDtypeStruct((NC, NP, D), jnp.float32),
            jax.ShapeDtypeStruct((NC, NP), jnp.float32),
        ],
        mesh=mesh,
        scratch_types=[
            pltpu.VMEM((NI, CH), jnp.int32),
            pltpu.VMEM((NI, CH), jnp.int32),
            pltpu.VMEM((NI, CH), jnp.float32),
            pltpu.VMEM((NB, CH, D), jnp.float32),
            pltpu.VMEM((CH,), jnp.float32),
            pltpu.VMEM((640,), jnp.float32),
            pltpu.VMEM_SHARED((NP, D), jnp.float32),
            pltpu.VMEM_SHARED((NP,), jnp.float32),
            [pltpu.SemaphoreType.DMA] * NB,
            [pltpu.SemaphoreType.DMA] * NB,
            [pltpu.SemaphoreType.DMA] * NB,
            [pltpu.SemaphoreType.DMA] * NB,
        ],
    )
    return kern(msgs, src, dst, w)


def _upd_body(x_ref, s_ref, c_ref, g_ref, bt_ref, w_ref, b_ref, o_ref):
    cnt = c_ref[0, :] + c_ref[1, :]
    agg = (s_ref[0] + s_ref[1]) / jnp.maximum(cnt, 1.0)[:, None]
    hx = x_ref[...] * (g_ref[pl.ds(0, D)] * RS) + bt_ref[pl.ds(0, D)]
    ha = agg * (g_ref[pl.ds(D, D)] * RS) + bt_ref[pl.ds(D, D)]
    acc = jnp.dot(hx, w_ref[pl.ds(0, D), :], preferred_element_type=jnp.float32)
    acc += jnp.dot(ha, w_ref[pl.ds(D, D), :], preferred_element_type=jnp.float32)
    o_ref[...] = _gelu(acc + b_ref[...])


def _update(x, sums, cnts, gamma2, beta2, W_upd, b_upd):
    BR = 512
    grid = (pl.cdiv(N, BR),)
    return pl.pallas_call(
        _upd_body,
        grid=grid,
        in_specs=[
            pl.BlockSpec((BR, D), lambda i: (i, 0)),
            pl.BlockSpec((NC, BR, D), lambda i: (0, i, 0)),
            pl.BlockSpec((NC, BR), lambda i: (0, i)),
            pl.BlockSpec((2 * D,), lambda i: (0,)),
            pl.BlockSpec((2 * D,), lambda i: (0,)),
            pl.BlockSpec((2 * D, D), lambda i: (0, 0)),
            pl.BlockSpec((D,), lambda i: (0,)),
        ],
        out_specs=pl.BlockSpec((BR, D), lambda i: (i, 0)),
        out_shape=jax.ShapeDtypeStruct((N, D), jnp.float32),
    )(x, sums, cnts, gamma2, beta2, W_upd, b_upd)


@jax.jit
def kernel(node_representation, edges, edge_weights, gamma1, beta1, W_prep,
           b_prep, gamma2, beta2, W_upd, b_upd):
    x = node_representation
    dst = edges[0].astype(jnp.int32)
    src = edges[1].astype(jnp.int32)
    w = edge_weights

    npad = EPAD - E
    src = jnp.concatenate([src, jnp.zeros((npad,), jnp.int32)])
    dst = jnp.concatenate([dst, jnp.full((npad,), N, jnp.int32)])
    w = jnp.concatenate([w, jnp.zeros((npad,), jnp.float32)])

    msgs = _prep_msgs(x, gamma1, beta1, W_prep, b_prep)
    sums, cnts = _sc_aggregate(msgs, src, dst, w)
    return _update(x, sums, cnts, gamma2, beta2, W_upd, b_upd)
